# baseline (speedup 1.0000x reference)
import functools

import jax
import jax.numpy as jnp
from jax import lax
from jax.experimental import pallas as pl
from jax.experimental.pallas import tpu as pltpu
from jax.experimental.pallas import tpu_sc as plsc

_DIM = 64
_N_TOK = 16384
_N_SAMPLES = 8192
_NW = 32
_IDS = _N_TOK + _N_SAMPLES
_VOCAB = 1000000
_SLAB = 1024
_NSLAB = 32
_NSLAB_G = _NW * _NSLAB
_RANGE = _NSLAB * _SLAB
_SHORT_LO = (_VOCAB // _SLAB) * _SLAB
_TAILW = _VOCAB - _SHORT_LO
_SEG = 1040
_STAGE = 96
_DUMP = _IDS
_OUT_ROWS = _IDS + 8


@functools.partial(
    pl.kernel,
    out_type=jax.ShapeDtypeStruct((_OUT_ROWS, 128), jnp.float32),
    mesh=plsc.VectorSubcoreMesh(core_axis_name="c", subcore_axis_name="s"),
    compiler_params=pltpu.CompilerParams(needs_layout_passes=False),
    scratch_types=[
        pltpu.VMEM((_SEG + 16,), jnp.int32),
        pltpu.VMEM((_SEG + 16,), jnp.int32),
        pltpu.VMEM((64,), jnp.int32),
        pltpu.VMEM((_DIM * _SLAB,), jnp.float32),
        pltpu.VMEM((_STAGE, 128), jnp.float32),
        pltpu.VMEM((_STAGE,), jnp.int32),
        pltpu.SemaphoreType.DMA,
        pltpu.SemaphoreType.DMA,
    ],
)
def _sc_scan_gather(sids_hbm, sord_hbm, starts_hbm, tab_t_hbm, tail_hbm,
                    out_hbm, wlid_v, wlord_v, starts_v, slab_v, stage_v,
                    sord_v, sem, sem2):
    wid = lax.axis_index("s") * 2 + lax.axis_index("c")
    lo_w = wid * _RANGE
    iota = lax.iota(jnp.int32, 16)
    dump16 = jnp.full((16,), _DUMP, jnp.int32)

    pltpu.sync_copy(starts_hbm.at[pl.ds(wid * _NSLAB, 48)],
                    starts_v.at[pl.ds(0, 48)])
    seg_lo = starts_v[pl.ds(0, 16)][0]
    seg_base = pl.multiple_of((seg_lo // 8) * 8, 8)
    pltpu.sync_copy(sids_hbm.at[pl.ds(seg_base, _SEG)],
                    wlid_v.at[pl.ds(0, _SEG)])
    pltpu.sync_copy(sord_hbm.at[pl.ds(seg_base, _SEG)],
                    wlord_v.at[pl.ds(0, _SEG)])

    def slab_loop(s, _):
        slab_lo = pl.multiple_of(lo_w + s * _SLAB, _SLAB)

        @pl.when(slab_lo + _SLAB <= _VOCAB)
        def _full():
            cps = [
                pltpu.async_copy(
                    tab_t_hbm.at[r, pl.ds(slab_lo, _SLAB)],
                    slab_v.at[pl.ds(r * _SLAB, _SLAB)],
                    sem,
                )
                for r in range(_DIM)
            ]
            for cp_ in cps:
                cp_.wait()

        @pl.when(slab_lo == _SHORT_LO)
        def _short():
            cps = [
                pltpu.async_copy(
                    tail_hbm.at[r],
                    slab_v.at[pl.ds(r * _SLAB, 640)],
                    sem,
                )
                for r in range(_DIM)
            ]
            for cp_ in cps:
                cp_.wait()

        sv = starts_v[pl.ds(s, 16)]
        st = sv[0] - seg_base
        n_s = sv[1] - sv[0]
        gtr = (n_s + 15) // 16

        for r in range(_STAGE // 16):
            sord_v[pl.ds(r * 16, 16)] = dump16

        def cp(g2, c):
            sord_v[pl.ds(g2 * 16, 16)] = wlord_v[pl.ds(st + g2 * 16, 16)]
            return c

        lax.fori_loop(0, gtr, cp, 0)
        sord_v[pl.ds(jnp.minimum(n_s, _STAGE - 16), 16)] = dump16

        def ext(g2, c):
            base = g2 * 16
            vcol = (wlid_v[pl.ds(st + base, 16)] - slab_lo) & (_SLAB - 1)
            for lane in range(16):
                widx = iota * _SLAB + vcol[lane]
                for r in range(_DIM // 16):
                    gv = plsc.load_gather(slab_v, [widx + (r * 16) * _SLAB])
                    stage_v[base + lane, pl.ds(r * 16, 16)] = gv
            return c

        lax.fori_loop(0, gtr, ext, 0)

        @pl.when(n_s > 0)
        def _flush():
            pltpu.async_copy(stage_v, out_hbm.at[sord_v], sem2).wait()

        return 0

    lax.fori_loop(0, _NSLAB, slab_loop, 0)


_TILE_N = 256
_GRID = _N_TOK // _TILE_N


def _tc_loss_body(x_ref, pe_ref, t_ref, nid_ref, ne_ref, acc_ref):
    i = pl.program_id(0)
    x = x_ref[...]
    z = lax.dot_general(
        x, ne_ref[...], (((1,), (1,)), ((), ())),
        preferred_element_type=jnp.float32,
    )
    t = t_ref[...]
    hits = nid_ref[0] == t
    z = jnp.where(hits, jnp.float32(-1e9), z)
    pos = jnp.sum(x * pe_ref[...], axis=1, keepdims=True)
    s = jnp.sum(jnp.exp(z), axis=1, keepdims=True) + jnp.exp(pos)
    loss = jnp.log(s) - pos
    mask = t != -100
    loss = jnp.where(mask, loss, jnp.float32(0.0))
    part = jnp.sum(loss, axis=(0, 1), keepdims=True)
    cnt = jnp.sum(mask.astype(jnp.float32), axis=(0, 1), keepdims=True)
    vec = jnp.concatenate([part, cnt], axis=1)

    @pl.when(i == 0)
    def _init():
        acc_ref[...] = jnp.zeros_like(acc_ref)

    acc_ref[...] += vec


_tc_loss = pl.pallas_call(
    _tc_loss_body,
    grid=(_GRID,),
    in_specs=[
        pl.BlockSpec((_TILE_N, _DIM), lambda i: (i, 0)),
        pl.BlockSpec((_TILE_N, _DIM), lambda i: (i, 0)),
        pl.BlockSpec((_TILE_N, 1), lambda i: (i, 0)),
        pl.BlockSpec((1, 1, _N_SAMPLES), lambda i: (0, 0, 0)),
        pl.BlockSpec((_N_SAMPLES, _DIM), lambda i: (0, 0)),
    ],
    out_specs=pl.BlockSpec((1, 2), lambda i: (0, 0)),
    out_shape=jax.ShapeDtypeStruct((1, 2), jnp.float32),
)


def kernel(inputs, target_ids, table, neg_ids):
    t = target_ids[:, 0]
    ids = jnp.concatenate([t, neg_ids])
    order = jnp.argsort(ids).astype(jnp.int32)
    sids = ids[order]
    edges = jnp.arange(_NSLAB_G + 1, dtype=jnp.int32) * _SLAB
    starts = jnp.searchsorted(sids, edges).astype(jnp.int32)
    starts_p = jnp.pad(starts, (0, (_NW - 1) * _NSLAB + 48 - _NSLAB_G - 1 + 8),
                       constant_values=_IDS)
    sids_p = jnp.pad(sids, (0, _SEG))
    sord_p = jnp.pad(order, (0, _SEG))

    table_t = table.T
    tail = jnp.pad(table_t[:, _SHORT_LO:], ((0, 0), (0, 640 - _TAILW)))
    rows = _sc_scan_gather(sids_p, sord_p, starts_p, table_t, tail)
    pos_emb = rows[:_N_TOK, :_DIM]
    neg_emb = rows[_N_TOK:_IDS, :_DIM]
    acc = _tc_loss(
        inputs, pos_emb, target_ids,
        neg_ids.reshape(1, 1, _N_SAMPLES), neg_emb,
    )
    loss = acc[0, 0] / acc[0, 1]
    return (jnp.asarray(0), loss)
